# Initial kernel scaffold; baseline (speedup 1.0000x reference)
import jax
import jax.numpy as jnp
from jax.experimental import pallas as pl


def kernel(x, gate_w, gate_b, W1, b1, W2, b2, W3, b3):
    raise NotImplementedError("write your pallas kernel here")


# speedup vs baseline: 1.2636x; 1.2636x over previous
import functools

import jax
import jax.numpy as jnp
from jax.experimental import pallas as pl

E = 8
K = 2
S = 2048
DIN = 1024
DOUT = 1024
TB = 256


def _gate_body(x_ref, gw_ref, gb_ref, wsel_ref, sel_ref):
    x = x_ref[:]
    logits = jnp.dot(x, gw_ref[:].T,
                     preferred_element_type=jnp.float32) + gb_ref[:][None, :]
    z = logits - jnp.max(logits, axis=0, keepdims=True)
    ez = jnp.exp(z)
    p = ez / jnp.sum(ez, axis=0, keepdims=True)
    cols = jax.lax.broadcasted_iota(jnp.int32, (S, E), 1)
    m1 = jnp.argmax(p, axis=1).astype(jnp.int32)
    p2 = jnp.where(cols == m1[:, None], -1.0, p)
    m2 = jnp.argmax(p2, axis=1).astype(jnp.int32)
    sel = jnp.logical_or(cols == m1[:, None], cols == m2[:, None])
    wsel_ref[:] = jnp.where(sel, p, 0.0)
    sel_ref[:] = sel.astype(jnp.float32)


def _expert_body(x_ref, W1_ref, b1_ref, W2_ref, b2_ref, W3_ref, b3_ref,
                 wsel_ref, V_ref):
    e = pl.program_id(0)
    tb = pl.program_id(1)

    @pl.when(jnp.logical_and(e == 0, tb == 0))
    def _init():
        V_ref[:] = jnp.zeros_like(V_ref)

    xb = x_ref[:]
    h = jnp.maximum(jnp.dot(xb, W1_ref[0].T, preferred_element_type=jnp.float32)
                    + b1_ref[0], 0.0)
    h = jnp.maximum(jnp.dot(h, W2_ref[0].T, preferred_element_type=jnp.float32)
                    + b2_ref[0], 0.0)
    h = jnp.maximum(jnp.dot(h, W3_ref[0].T, preferred_element_type=jnp.float32)
                    + b3_ref[0], 0.0)
    onehot = (jax.lax.broadcasted_iota(jnp.int32, (E, 1), 0) == e
              ).astype(jnp.float32)
    wcol = jnp.dot(wsel_ref[:], onehot)
    vpart = jnp.sum(h * wcol, axis=0, keepdims=True)
    V_ref[:] += onehot * vpart


def _combine_body(sel_ref, V_ref, out_ref):
    out_ref[:] = jnp.dot(sel_ref[:], V_ref[:],
                         preferred_element_type=jnp.float32)


@functools.partial(jax.jit, static_argnames=())
def kernel(x, gate_w, gate_b, W1, b1, W2, b2, W3, b3):
    B = x.shape[0]
    x2 = x.reshape(B * S, DIN)

    wsel, sel = pl.pallas_call(
        _gate_body,
        out_shape=(jax.ShapeDtypeStruct((S, E), jnp.float32),
                   jax.ShapeDtypeStruct((S, E), jnp.float32)),
    )(x2, gate_w, gate_b)

    V = pl.pallas_call(
        _expert_body,
        grid=(E, S // TB),
        in_specs=[
            pl.BlockSpec((TB, DIN), lambda e, tb: (tb, 0)),
            pl.BlockSpec((1, DOUT, DIN), lambda e, tb: (e, 0, 0)),
            pl.BlockSpec((1, 1, DOUT), lambda e, tb: (e, 0, 0)),
            pl.BlockSpec((1, DOUT, DOUT), lambda e, tb: (e, 0, 0)),
            pl.BlockSpec((1, 1, DOUT), lambda e, tb: (e, 0, 0)),
            pl.BlockSpec((1, DOUT, DOUT), lambda e, tb: (e, 0, 0)),
            pl.BlockSpec((1, 1, DOUT), lambda e, tb: (e, 0, 0)),
            pl.BlockSpec((TB, E), lambda e, tb: (tb, 0)),
        ],
        out_specs=pl.BlockSpec((E, DOUT), lambda e, tb: (0, 0)),
        out_shape=jax.ShapeDtypeStruct((E, DOUT), jnp.float32),
    )(x2, W1, b1.reshape(E, 1, DOUT), W2, b2.reshape(E, 1, DOUT),
      W3, b3.reshape(E, 1, DOUT), wsel)

    out = pl.pallas_call(
        _combine_body,
        grid=(S // TB,),
        in_specs=[
            pl.BlockSpec((TB, E), lambda tb: (tb, 0)),
            pl.BlockSpec((E, DOUT), lambda tb: (0, 0)),
        ],
        out_specs=pl.BlockSpec((TB, DOUT), lambda tb: (tb, 0)),
        out_shape=jax.ShapeDtypeStruct((S, DOUT), jnp.float32),
    )(sel, V)

    return out.reshape(B, S, DOUT)
